# baseline (speedup 1.0000x reference)
import functools

import jax
import jax.numpy as jnp
from jax import lax
from jax.experimental import pallas as pl
from jax.experimental.pallas import tpu as pltpu
from jax.experimental.pallas import tpu_sc as plsc

N = 10000
D = 128
G = 64
NC = 2
NS = 16
NW = NC * NS
C = 80
ACC_ROWS = 10112
ZROWS = ACC_ROWS // NS
OROWS = ACC_ROWS // NS
BLK = 1000
NBLK = N // BLK


NBUF = 4
GROUP = 8
STEP = 2 * GROUP


def _make_aggregate(nchunk):
    assert nchunk % STEP == 0 and nchunk >= 2 * STEP
    nit = nchunk // STEP
    ngr = nchunk // GROUP

    @functools.partial(
        pl.kernel,
        out_type=jax.ShapeDtypeStruct((NC, ACC_ROWS, D), jnp.float32),
        mesh=plsc.VectorSubcoreMesh(core_axis_name="c", subcore_axis_name="s"),
        scratch_types=[
            pltpu.VMEM((2, GROUP, C), jnp.int32),
            pltpu.VMEM((2, GROUP, C), jnp.int32),
            pltpu.VMEM((NBUF, C, D), jnp.float32),
            pltpu.VMEM_SHARED((ACC_ROWS, D), jnp.float32),
        ]
        + [pltpu.SemaphoreType.DMA] * (2 * NBUF + 1),
    )
    def agg(y_hbm, src_hbm, dst_hbm, zero_hbm, out_hbm, src_ring, dst_ring,
            bufs, acc, *sems):
        c = lax.axis_index("c")
        s = lax.axis_index("s")
        gsems = sems[:NBUF]
        ssems = sems[NBUF:2 * NBUF]
        isem = sems[2 * NBUF]

        def issue_prefetch(start, ring_idx):
            sl = pl.ds(start, GROUP)
            pltpu.async_copy(src_hbm.at[c, s, sl], src_ring.at[ring_idx], isem)
            pltpu.async_copy(dst_hbm.at[c, s, sl], dst_ring.at[ring_idx], isem)

        def wait_prefetch():
            pltpu.make_async_copy(
                src_hbm.at[c, s, pl.ds(0, GROUP)], src_ring.at[0], isem).wait()
            pltpu.make_async_copy(
                src_hbm.at[c, s, pl.ds(0, GROUP)], dst_ring.at[0], isem).wait()

        def issue_gather(ring_idx, row, b):
            pltpu.async_copy(y_hbm.at[src_ring.at[ring_idx, row]],
                             bufs.at[b], gsems[b])

        def wait_gather(b):
            pltpu.make_async_copy(
                zero_hbm.at[pl.ds(0, C)], bufs.at[b], gsems[b]).wait()

        def issue_scatter(ring_idx, row, b):
            pltpu.async_copy(bufs.at[b], acc.at[dst_ring.at[ring_idx, row]],
                             ssems[b], add=True)

        def wait_scatter(ring_idx, row, b):
            pltpu.make_async_copy(
                bufs.at[b], acc.at[dst_ring.at[ring_idx, row]],
                ssems[b]).wait()

        issue_prefetch(0, 0)
        wait_prefetch()
        issue_gather(0, 0, 0)
        issue_gather(0, 1, 1)
        pltpu.sync_copy(zero_hbm, acc.at[pl.ds(s * ZROWS, ZROWS)])
        plsc.subcore_barrier()

        def body(i, carry):
            base = pl.multiple_of(i * STEP, STEP)
            for half in (0, 1):
                for b in range(GROUP):
                    bj = b % NBUF
                    b2 = (b + 2) % NBUF
                    wait_gather(bj)
                    issue_scatter(half, b, bj)
                    if b >= 2:
                        wait_scatter(half, b - 2, b2)
                    elif half == 1:
                        wait_scatter(0, GROUP - 2 + b, b2)
                    else:
                        @pl.when(i > 0)
                        def _():
                            wait_scatter(1, GROUP - 2 + b, b2)
                    if b == GROUP - 2 and half == 0:
                        wait_prefetch()
                    if b < GROUP - 2:
                        issue_gather(half, b + 2, b2)
                    elif half == 0:
                        issue_gather(1, b - (GROUP - 2), b2)
                    else:
                        @pl.when(i < nit - 1)
                        def _():
                            if b == GROUP - 2:
                                wait_prefetch()
                            issue_gather(0, b - (GROUP - 2), b2)
                    if b == 1:
                        if half == 0:
                            issue_prefetch(base + GROUP, 1)
                        else:
                            @pl.when(i < nit - 1)
                            def _():
                                issue_prefetch(base + 2 * GROUP, 0)
            return carry

        lax.fori_loop(0, nit, body, 0)
        wait_scatter(1, GROUP - 2, (GROUP - 2) % NBUF)
        wait_scatter(1, GROUP - 1, (GROUP - 1) % NBUF)
        plsc.subcore_barrier()
        pltpu.sync_copy(acc.at[pl.ds(s * OROWS, OROWS)],
                        out_hbm.at[c, pl.ds(s * OROWS, OROWS)])

    return agg


def _lin_body(x_ref, p_ref, w_ref, b_ref, o_ref):
    xx = x_ref[...] + p_ref[0] + p_ref[1]
    o_ref[...] = (
        jnp.dot(xx, w_ref[...], preferred_element_type=jnp.float32) + b_ref[...]
    )


def _linear(x, p, W, b):
    return pl.pallas_call(
        _lin_body,
        grid=(NBLK,),
        in_specs=[
            pl.BlockSpec((BLK, D), lambda i: (i, 0)),
            pl.BlockSpec((NC, BLK, D), lambda i: (0, i, 0)),
            pl.BlockSpec((D, D), lambda i: (0, 0)),
            pl.BlockSpec((1, D), lambda i: (0, 0)),
        ],
        out_specs=pl.BlockSpec((BLK, D), lambda i: (i, 0)),
        out_shape=jax.ShapeDtypeStruct((N, D), jnp.float32),
    )(x, p, W, b)


def _final_body(h_ref, p_ref, w2_ref, b2_ref, bt_ref, wc1_ref, bc1_ref,
                wc2_ref, bc2_ref, o_ref, pooled):
    i = pl.program_id(0)
    h2 = (
        jnp.dot(h_ref[...] + p_ref[0] + p_ref[1], w2_ref[...],
                preferred_element_type=jnp.float32) + b2_ref[...]
    )
    bids = bt_ref[0]
    onehot_t = (lax.broadcasted_iota(jnp.int32, (G, BLK), 0) == bids).astype(
        jnp.float32)
    contrib = jnp.dot(onehot_t, h2, preferred_element_type=jnp.float32)

    @pl.when(i == 0)
    def _():
        pooled[...] = contrib

    @pl.when(i > 0)
    def _():
        pooled[...] += contrib

    @pl.when(i == NBLK - 1)
    def _():
        hid = jnp.maximum(
            jnp.dot(pooled[...], wc1_ref[...],
                    preferred_element_type=jnp.float32) + bc1_ref[...], 0.0)
        score = (
            jnp.dot(hid, wc2_ref[...], preferred_element_type=jnp.float32)
            + bc2_ref[...]
        )
        o_ref[...] = 1.0 / (1.0 + jnp.exp(-score))


def _final(h, p, W2, b2, batch_r, Wc1, bc1, Wc2, bc2):
    return pl.pallas_call(
        _final_body,
        grid=(NBLK,),
        in_specs=[
            pl.BlockSpec((BLK, D), lambda i: (i, 0)),
            pl.BlockSpec((NC, BLK, D), lambda i: (0, i, 0)),
            pl.BlockSpec((D, D), lambda i: (0, 0)),
            pl.BlockSpec((1, D), lambda i: (0, 0)),
            pl.BlockSpec((1, 1, BLK), lambda i: (i, 0, 0)),
            pl.BlockSpec((D, G), lambda i: (0, 0)),
            pl.BlockSpec((1, G), lambda i: (0, 0)),
            pl.BlockSpec((G, 1), lambda i: (0, 0)),
            pl.BlockSpec((1, 1), lambda i: (0, 0)),
        ],
        out_specs=pl.BlockSpec((G, 1), lambda i: (0, 0)),
        out_shape=jax.ShapeDtypeStruct((G, 1), jnp.float32),
        scratch_shapes=[pltpu.VMEM((G, D), jnp.float32)],
    )(h, p, W2, b2, batch_r, Wc1, bc1, Wc2, bc2)


def kernel(joint_x, joint_edge_index, joint_batch, W1, b1, W2, b2,
           Wc1, bc1, Wc2, bc2):
    x = joint_x.astype(jnp.float32)
    ei = joint_edge_index.astype(jnp.int32)
    src, dst = ei[0], ei[1]
    e = src.shape[0]
    nchunk = -(-e // (NW * C))
    nchunk = max(-(-nchunk // STEP) * STEP, 2 * STEP)
    epad = NW * nchunk * C
    pad = epad - e
    if pad:
        pad_i = jnp.arange(pad, dtype=jnp.int32)
        src = jnp.concatenate([src, pad_i % N])
        dst = jnp.concatenate([dst, N + pad_i % (ACC_ROWS - N)])
    src_r = src.reshape(NC, NS, nchunk, C)
    dst_r = dst.reshape(NC, NS, nchunk, C)
    zeros = jnp.zeros((ZROWS, D), jnp.float32)

    agg = _make_aggregate(nchunk)
    p1 = agg(x, src_r, dst_r, zeros)
    h1 = _linear(x, p1, W1, b1.reshape(1, D))
    p2 = agg(h1, src_r, dst_r, zeros)
    batch_r = joint_batch.astype(jnp.int32).reshape(NBLK, 1, BLK)
    out = _final(h1, p2, W2, b2.reshape(1, D), batch_r,
                 Wc1, bc1.reshape(1, G), Wc2, bc2.reshape(1, 1))
    return out.reshape(G)


# speedup vs baseline: 1.0986x; 1.0986x over previous
import functools

import jax
import jax.numpy as jnp
from jax import lax
from jax.experimental import pallas as pl
from jax.experimental.pallas import tpu as pltpu
from jax.experimental.pallas import tpu_sc as plsc

N = 10000
D = 128
G = 64
NC = 2
NS = 16
NW = NC * NS
C = 80
ACC_ROWS = 10112
ZROWS = ACC_ROWS // NS
OROWS = ACC_ROWS // NS
BLK = 1000
NBLK = N // BLK


NBUF = 4
GROUP = 8
STEP = 2 * GROUP


def _make_aggregate(nchunk):
    assert nchunk % STEP == 0 and nchunk >= 2 * STEP
    nit = nchunk // STEP
    ngr = nchunk // GROUP

    @functools.partial(
        pl.kernel,
        out_type=jax.ShapeDtypeStruct((NC, ACC_ROWS, D), jnp.float32),
        mesh=plsc.VectorSubcoreMesh(core_axis_name="c", subcore_axis_name="s"),
        scratch_types=[
            pltpu.VMEM((2, GROUP, C), jnp.int32),
            pltpu.VMEM((2, GROUP, C), jnp.int32),
            pltpu.VMEM((NBUF, C, D), jnp.float32),
            pltpu.VMEM_SHARED((ACC_ROWS, D), jnp.float32),
        ]
        + [pltpu.SemaphoreType.DMA] * (2 * NBUF + 1),
    )
    def agg(y_hbm, src_hbm, dst_hbm, zero_hbm, out_hbm, src_ring, dst_ring,
            bufs, acc, *sems):
        c = lax.axis_index("c")
        s = lax.axis_index("s")
        gsems = sems[:NBUF]
        ssems = sems[NBUF:2 * NBUF]
        isem = sems[2 * NBUF]

        def issue_prefetch(start, ring_idx):
            sl = pl.ds(start, GROUP)
            pltpu.async_copy(src_hbm.at[c, s, sl], src_ring.at[ring_idx], isem)
            pltpu.async_copy(dst_hbm.at[c, s, sl], dst_ring.at[ring_idx], isem)

        def wait_prefetch():
            pltpu.make_async_copy(
                src_hbm.at[c, s, pl.ds(0, GROUP)], src_ring.at[0], isem).wait()
            pltpu.make_async_copy(
                src_hbm.at[c, s, pl.ds(0, GROUP)], dst_ring.at[0], isem).wait()

        def issue_gather(ring_idx, row, b):
            pltpu.async_copy(y_hbm.at[src_ring.at[ring_idx, row]],
                             bufs.at[b], gsems[b])

        def wait_gather(b):
            pltpu.make_async_copy(
                zero_hbm.at[pl.ds(0, C)], bufs.at[b], gsems[b]).wait()

        def issue_scatter(ring_idx, row, b):
            pltpu.async_copy(bufs.at[b], acc.at[dst_ring.at[ring_idx, row]],
                             ssems[b], add=True)

        def wait_scatter(ring_idx, row, b):
            pltpu.make_async_copy(
                bufs.at[b], acc.at[dst_ring.at[ring_idx, row]],
                ssems[b]).wait()

        issue_prefetch(0, 0)
        wait_prefetch()
        issue_gather(0, 0, 0)
        issue_gather(0, 1, 1)
        pltpu.sync_copy(zero_hbm, acc.at[pl.ds(s * ZROWS, ZROWS)])
        plsc.subcore_barrier()

        def body(i, carry):
            base = pl.multiple_of(i * STEP, STEP)
            for half in (0, 1):
                for b in range(GROUP):
                    bj = b % NBUF
                    b2 = (b + 2) % NBUF
                    if b >= 2:
                        wait_scatter(half, b - 2, b2)
                    elif half == 1:
                        wait_scatter(0, GROUP - 2 + b, b2)
                    else:
                        @pl.when(i > 0)
                        def _():
                            wait_scatter(1, GROUP - 2 + b, b2)
                    if b == GROUP - 2 and half == 0:
                        wait_prefetch()
                    if b < GROUP - 2:
                        issue_gather(half, b + 2, b2)
                    elif half == 0:
                        issue_gather(1, b - (GROUP - 2), b2)
                    else:
                        @pl.when(i < nit - 1)
                        def _():
                            if b == GROUP - 2:
                                wait_prefetch()
                            issue_gather(0, b - (GROUP - 2), b2)
                    wait_gather(bj)
                    issue_scatter(half, b, bj)
                    if b == 1:
                        if half == 0:
                            issue_prefetch(base + GROUP, 1)
                        else:
                            @pl.when(i < nit - 1)
                            def _():
                                issue_prefetch(base + 2 * GROUP, 0)
            return carry

        lax.fori_loop(0, nit, body, 0)
        wait_scatter(1, GROUP - 2, (GROUP - 2) % NBUF)
        wait_scatter(1, GROUP - 1, (GROUP - 1) % NBUF)
        plsc.subcore_barrier()
        pltpu.sync_copy(acc.at[pl.ds(s * OROWS, OROWS)],
                        out_hbm.at[c, pl.ds(s * OROWS, OROWS)])

    return agg


def _lin_body(x_ref, p_ref, w_ref, b_ref, o_ref):
    xx = x_ref[...] + p_ref[0] + p_ref[1]
    o_ref[...] = (
        jnp.dot(xx, w_ref[...], preferred_element_type=jnp.float32) + b_ref[...]
    )


def _linear(x, p, W, b):
    return pl.pallas_call(
        _lin_body,
        grid=(NBLK,),
        in_specs=[
            pl.BlockSpec((BLK, D), lambda i: (i, 0)),
            pl.BlockSpec((NC, BLK, D), lambda i: (0, i, 0)),
            pl.BlockSpec((D, D), lambda i: (0, 0)),
            pl.BlockSpec((1, D), lambda i: (0, 0)),
        ],
        out_specs=pl.BlockSpec((BLK, D), lambda i: (i, 0)),
        out_shape=jax.ShapeDtypeStruct((N, D), jnp.float32),
    )(x, p, W, b)


def _final_body(h_ref, p_ref, w2_ref, b2_ref, bt_ref, wc1_ref, bc1_ref,
                wc2_ref, bc2_ref, o_ref, pooled):
    i = pl.program_id(0)
    h2 = (
        jnp.dot(h_ref[...] + p_ref[0] + p_ref[1], w2_ref[...],
                preferred_element_type=jnp.float32) + b2_ref[...]
    )
    bids = bt_ref[0]
    onehot_t = (lax.broadcasted_iota(jnp.int32, (G, BLK), 0) == bids).astype(
        jnp.float32)
    contrib = jnp.dot(onehot_t, h2, preferred_element_type=jnp.float32)

    @pl.when(i == 0)
    def _():
        pooled[...] = contrib

    @pl.when(i > 0)
    def _():
        pooled[...] += contrib

    @pl.when(i == NBLK - 1)
    def _():
        hid = jnp.maximum(
            jnp.dot(pooled[...], wc1_ref[...],
                    preferred_element_type=jnp.float32) + bc1_ref[...], 0.0)
        score = (
            jnp.dot(hid, wc2_ref[...], preferred_element_type=jnp.float32)
            + bc2_ref[...]
        )
        o_ref[...] = 1.0 / (1.0 + jnp.exp(-score))


def _final(h, p, W2, b2, batch_r, Wc1, bc1, Wc2, bc2):
    return pl.pallas_call(
        _final_body,
        grid=(NBLK,),
        in_specs=[
            pl.BlockSpec((BLK, D), lambda i: (i, 0)),
            pl.BlockSpec((NC, BLK, D), lambda i: (0, i, 0)),
            pl.BlockSpec((D, D), lambda i: (0, 0)),
            pl.BlockSpec((1, D), lambda i: (0, 0)),
            pl.BlockSpec((1, 1, BLK), lambda i: (i, 0, 0)),
            pl.BlockSpec((D, G), lambda i: (0, 0)),
            pl.BlockSpec((1, G), lambda i: (0, 0)),
            pl.BlockSpec((G, 1), lambda i: (0, 0)),
            pl.BlockSpec((1, 1), lambda i: (0, 0)),
        ],
        out_specs=pl.BlockSpec((G, 1), lambda i: (0, 0)),
        out_shape=jax.ShapeDtypeStruct((G, 1), jnp.float32),
        scratch_shapes=[pltpu.VMEM((G, D), jnp.float32)],
    )(h, p, W2, b2, batch_r, Wc1, bc1, Wc2, bc2)


def kernel(joint_x, joint_edge_index, joint_batch, W1, b1, W2, b2,
           Wc1, bc1, Wc2, bc2):
    x = joint_x.astype(jnp.float32)
    ei = joint_edge_index.astype(jnp.int32)
    src, dst = ei[0], ei[1]
    e = src.shape[0]
    nchunk = -(-e // (NW * C))
    nchunk = max(-(-nchunk // STEP) * STEP, 2 * STEP)
    epad = NW * nchunk * C
    pad = epad - e
    if pad:
        pad_i = jnp.arange(pad, dtype=jnp.int32)
        src = jnp.concatenate([src, pad_i % N])
        dst = jnp.concatenate([dst, N + pad_i % (ACC_ROWS - N)])
    src_r = src.reshape(NC, NS, nchunk, C)
    dst_r = dst.reshape(NC, NS, nchunk, C)
    zeros = jnp.zeros((ZROWS, D), jnp.float32)

    agg = _make_aggregate(nchunk)
    p1 = agg(x, src_r, dst_r, zeros)
    h1 = _linear(x, p1, W1, b1.reshape(1, D))
    p2 = agg(h1, src_r, dst_r, zeros)
    batch_r = joint_batch.astype(jnp.int32).reshape(NBLK, 1, BLK)
    out = _final(h1, p2, W2, b2.reshape(1, D), batch_r,
                 Wc1, bc1.reshape(1, G), Wc2, bc2.reshape(1, 1))
    return out.reshape(G)
